# Initial kernel scaffold; baseline (speedup 1.0000x reference)
import jax
import jax.numpy as jnp
from jax.experimental import pallas as pl


def kernel(in_feat, edge_index, W_conv, b_conv, W_lin, b_lin):
    raise NotImplementedError("write your pallas kernel here")


# speedup vs baseline: 3.8754x; 3.8754x over previous
import functools

import jax
import jax.numpy as jnp
from jax import lax
from jax.experimental import pallas as pl
from jax.experimental.pallas import tpu as pltpu
from jax.experimental.pallas import tpu_sc as plsc

NC = 2
NS = 16
L = 16
NW = NC * NS

N_NODES = 10000
N_EDGES = 320000
NP = 10240
F = 128
K = 80

_MESH = plsc.VectorSubcoreMesh(
    core_axis_name="c", subcore_axis_name="s", num_cores=NC, num_subcores=NS)


@functools.partial(
    pl.kernel,
    out_type=jax.ShapeDtypeStruct((4, NP), jnp.float32),
    mesh=_MESH,
    scratch_types=[
        pltpu.VMEM((K,), jnp.int32),
        pltpu.VMEM((K,), jnp.float32),
        pltpu.VMEM((K,), jnp.float32),
        pltpu.VMEM_SHARED((NP,), jnp.float32),
        pltpu.VMEM_SHARED((NP,), jnp.float32),
    ],
)
def _deg_kernel(src_hbm, dst_hbm, out_hbm, idx_v, ones_v, zeros_v,
                acc_out, acc_in):
    cid = lax.axis_index("c")
    sid = lax.axis_index("s")
    wid = sid * NC + cid

    def _fill(i, _):
        ones_v[pl.ds(i * L, L)] = jnp.ones((L,), jnp.float32)
        zeros_v[pl.ds(i * L, L)] = jnp.zeros((L,), jnp.float32)
        return _
    lax.fori_loop(0, K // L, _fill, None)

    col0 = sid * (NP // NS)
    def _zero(k, _):
        pltpu.sync_copy(zeros_v, acc_out.at[pl.ds(col0 + k * K, K)])
        pltpu.sync_copy(zeros_v, acc_in.at[pl.ds(col0 + k * K, K)])
        return _
    lax.fori_loop(0, (NP // NS) // K, _zero, None)
    plsc.subcore_barrier()

    ew = N_EDGES // NW
    base = wid * ew
    def _scat(j, _):
        off = base + j * K
        pltpu.sync_copy(src_hbm.at[pl.ds(off, K)], idx_v)
        pltpu.sync_copy(ones_v, acc_out.at[idx_v], add=True)
        pltpu.sync_copy(dst_hbm.at[pl.ds(off, K)], idx_v)
        pltpu.sync_copy(ones_v, acc_in.at[idx_v], add=True)
        return _
    lax.fori_loop(0, ew // K, _scat, None)
    plsc.subcore_barrier()

    nc = NP // NS
    pltpu.sync_copy(acc_out.at[pl.ds(col0, nc)],
                    out_hbm.at[2 * cid + 0, pl.ds(col0, nc)])
    pltpu.sync_copy(acc_in.at[pl.ds(col0, nc)],
                    out_hbm.at[2 * cid + 1, pl.ds(col0, nc)])


def _scale_mm_body(x_ref, d_ref, w_ref, o_ref):
    d = d_ref[...]
    s = lax.rsqrt(jnp.maximum(d[:, 0:1] + d[:, 2:3], 1.0))
    o_ref[...] = jnp.dot(x_ref[...] * s, w_ref[...],
                         preferred_element_type=jnp.float32)


@functools.partial(
    pl.kernel,
    out_type=jax.ShapeDtypeStruct((NC, NP, F), jnp.float32),
    mesh=_MESH,
    scratch_types=[
        pltpu.VMEM((K,), jnp.int32),
        pltpu.VMEM((K,), jnp.int32),
        pltpu.VMEM((K, F), jnp.float32),
        pltpu.VMEM_SHARED((NP, F), jnp.float32),
        pltpu.SemaphoreType.DMA,
    ],
)
def _edge_kernel(z_hbm, src_hbm, dst_hbm, out_hbm, idx_s, idx_d, rows_v,
                 acc_sh, sem):
    cid = lax.axis_index("c")
    sid = lax.axis_index("s")
    wid = sid * NC + cid

    def _fill(i, _):
        r = i // (F // L)
        c = lax.rem(i, F // L)
        rows_v[r, pl.ds(c * L, L)] = jnp.zeros((L,), jnp.float32)
        return _
    lax.fori_loop(0, K * (F // L), _fill, None)
    row0 = sid * (NP // NS)
    def _zero(k, _):
        pltpu.sync_copy(rows_v, acc_sh.at[pl.ds(row0 + k * K, K), :])
        return _
    lax.fori_loop(0, (NP // NS) // K, _zero, None)
    plsc.subcore_barrier()

    ew = N_EDGES // NW
    base = wid * ew
    def _body(j, _):
        off = base + j * K
        pltpu.sync_copy(src_hbm.at[pl.ds(off, K)], idx_s)
        cp = pltpu.async_copy(z_hbm.at[idx_s], rows_v, sem)
        pltpu.sync_copy(dst_hbm.at[pl.ds(off, K)], idx_d)
        cp.wait()
        pltpu.sync_copy(rows_v, acc_sh.at[idx_d], add=True)
        return _
    lax.fori_loop(0, ew // K, _body, None)
    plsc.subcore_barrier()

    nr = NP // NS
    pltpu.sync_copy(acc_sh.at[pl.ds(row0, nr), :],
                    out_hbm.at[cid, pl.ds(row0, nr), :])


def _final_body(p_ref, d_ref, bc_ref, wl_ref, bl_ref, o_ref, acc_ref):
    i = pl.program_id(0)
    nb = pl.num_programs(0)

    @pl.when(i == 0)
    def _():
        acc_ref[...] = jnp.zeros_like(acc_ref)

    blk = p_ref.shape[1]
    p = p_ref[0] + p_ref[1]
    d = d_ref[...]
    s = lax.rsqrt(jnp.maximum(d[:, 1:2] + d[:, 3:4], 1.0))
    h = jnp.maximum(p * s + bc_ref[...], 0.0)
    rows = i * blk + lax.broadcasted_iota(jnp.int32, (blk, 1), 0)
    h = jnp.where(rows < N_NODES, h, 0.0)
    acc_ref[...] += jnp.sum(h, axis=0, keepdims=True)

    @pl.when(i == nb - 1)
    def _():
        m = acc_ref[...] / float(N_NODES)
        logits = lax.dot_general(m, wl_ref[...], (((1,), (1,)), ((), ())),
                                 preferred_element_type=jnp.float32)
        logits = logits + bl_ref[...]
        e = jnp.exp(logits - jnp.max(logits))
        o_ref[...] = e / jnp.sum(e)


def kernel(in_feat, edge_index, W_conv, b_conv, W_lin, b_lin):
    src = edge_index[0].astype(jnp.int32)
    dst = edge_index[1].astype(jnp.int32)

    deg = _deg_kernel(src, dst)
    deg_t = deg.T

    z = pl.pallas_call(
        _scale_mm_body,
        grid=(25,),
        in_specs=[
            pl.BlockSpec((400, F), lambda i: (i, 0)),
            pl.BlockSpec((400, 4), lambda i: (i, 0)),
            pl.BlockSpec((F, F), lambda i: (0, 0)),
        ],
        out_specs=pl.BlockSpec((400, F), lambda i: (i, 0)),
        out_shape=jax.ShapeDtypeStruct((N_NODES, F), jnp.float32),
    )(in_feat, deg_t[:N_NODES], W_conv)

    partials = _edge_kernel(z, src, dst)

    BLK = 512
    p = pl.pallas_call(
        _final_body,
        grid=(NP // BLK,),
        in_specs=[
            pl.BlockSpec((NC, BLK, F), lambda i: (0, i, 0)),
            pl.BlockSpec((BLK, 4), lambda i: (i, 0)),
            pl.BlockSpec((1, F), lambda i: (0, 0)),
            pl.BlockSpec(W_lin.shape, lambda i: (0, 0)),
            pl.BlockSpec((1, W_lin.shape[0]), lambda i: (0, 0)),
        ],
        out_specs=pl.BlockSpec((1, W_lin.shape[0]), lambda i: (0, 0)),
        out_shape=jax.ShapeDtypeStruct((1, W_lin.shape[0]), jnp.float32),
        scratch_shapes=[pltpu.VMEM((1, F), jnp.float32)],
    )(partials, deg_t, b_conv.reshape(1, F), W_lin,
      b_lin.reshape(1, W_lin.shape[0]))

    return p.reshape(W_lin.shape[0])


# speedup vs baseline: 4.8288x; 1.2460x over previous
import functools

import jax
import jax.numpy as jnp
from jax import lax
from jax.experimental import pallas as pl
from jax.experimental.pallas import tpu as pltpu
from jax.experimental.pallas import tpu_sc as plsc

NC = 2
NS = 16
L = 16
NW = NC * NS

N_NODES = 10000
N_EDGES = 320000
NP = 10240
F = 128
K = 80
EW = N_EDGES // NW
KE = 128
CH = 79
E_PAD = NW * CH * KE
IR = 4
RB = 2

_MESH = plsc.VectorSubcoreMesh(
    core_axis_name="c", subcore_axis_name="s", num_cores=NC, num_subcores=NS)


@functools.partial(
    pl.kernel,
    out_type=jax.ShapeDtypeStruct((4, NP), jnp.float32),
    mesh=_MESH,
    scratch_types=[
        pltpu.VMEM((EW,), jnp.int32),
        pltpu.VMEM((NP,), jnp.float32),
        pltpu.VMEM((NP,), jnp.float32),
        pltpu.VMEM((NS, 1, NP // NS), jnp.float32),
        pltpu.VMEM((NP // NS,), jnp.float32),
        pltpu.VMEM_SHARED((NS, 2, 1, NP), jnp.float32),
    ],
    compiler_params=pltpu.CompilerParams(needs_layout_passes=False),
)
def _deg_kernel(src_hbm, dst_hbm, out_hbm, idx_v, h_out, h_in, rbuf, rres,
                stage):
    cid = lax.axis_index("c")
    sid = lax.axis_index("s")
    wid = sid * NC + cid
    base = wid * EW
    ones = jnp.ones((L,), jnp.float32)
    zeros = jnp.zeros((L,), jnp.float32)

    def _zero(i, _):
        h_out[pl.ds(i * L, L)] = zeros
        h_in[pl.ds(i * L, L)] = zeros
        return _
    lax.fori_loop(0, NP // L, _zero, None)

    pltpu.sync_copy(src_hbm.at[pl.ds(base, EW)], idx_v)
    def _hist_out(i, _):
        plsc.addupdate_scatter(h_out, [idx_v[pl.ds(i * L, L)]], ones)
        return _
    lax.fori_loop(0, EW // L, _hist_out, None)

    pltpu.sync_copy(dst_hbm.at[pl.ds(base, EW)], idx_v)
    def _hist_in(i, _):
        plsc.addupdate_scatter(h_in, [idx_v[pl.ds(i * L, L)]], ones)
        return _
    lax.fori_loop(0, EW // L, _hist_in, None)

    pltpu.sync_copy(h_out, stage.at[sid, 0, 0, :])
    pltpu.sync_copy(h_in, stage.at[sid, 1, 0, :])
    plsc.subcore_barrier()

    nc_ = NP // NS
    col0 = sid * nc_
    for r in range(2):
        def _pull(t, _):
            pltpu.sync_copy(stage.at[t, r, 0, pl.ds(col0, nc_)], rbuf.at[t, 0])
            return _
        lax.fori_loop(0, NS, _pull, None)
        def _red(i, _):
            acc = rbuf[0, 0, pl.ds(i * L, L)]
            for t in range(1, NS):
                acc = acc + rbuf[t, 0, pl.ds(i * L, L)]
            rres[pl.ds(i * L, L)] = acc
            return _
        lax.fori_loop(0, nc_ // L, _red, None)
        pltpu.sync_copy(rres, out_hbm.at[2 * cid + r, pl.ds(col0, nc_)])


def _scale_mm_body(x_ref, d_ref, w_ref, o_ref):
    d = d_ref[...]
    s = lax.rsqrt(jnp.maximum(d[:, 0:1] + d[:, 2:3], 1.0))
    o_ref[...] = jnp.dot(x_ref[...] * s, w_ref[...],
                         preferred_element_type=jnp.float32)


@functools.partial(
    pl.kernel,
    out_type=jax.ShapeDtypeStruct((NC, NP, F), jnp.float32),
    mesh=_MESH,
    scratch_types=[
        pltpu.VMEM((IR, 2, KE), jnp.int32),
        pltpu.VMEM((RB, KE, F), jnp.float32),
        pltpu.VMEM_SHARED((NP, F), jnp.float32),
        pltpu.SemaphoreType.DMA((IR,)),
        pltpu.SemaphoreType.DMA((RB,)),
        pltpu.SemaphoreType.DMA((RB,)),
    ],
)
def _edge_kernel(z_hbm, idx_hbm, out_hbm, idx_r, rows_v, acc_sh,
                 isem, gsem, ssem):
    cid = lax.axis_index("c")
    sid = lax.axis_index("s")
    wid = sid * NC + cid
    base_c = wid * CH

    def _fill(i, _):
        r = i // (F // L)
        c = lax.rem(i, F // L)
        rows_v[0, r, pl.ds(c * L, L)] = jnp.zeros((L,), jnp.float32)
        return _
    lax.fori_loop(0, KE * (F // L), _fill, None)
    row0 = sid * (NP // NS)
    def _zero(k, _):
        pltpu.sync_copy(rows_v.at[0], acc_sh.at[pl.ds(row0 + k * KE, KE), :])
        return _
    lax.fori_loop(0, (NP // NS) // KE, _zero, None)
    plsc.subcore_barrier()

    def _idx_load(c):
        pltpu.async_copy(idx_hbm.at[base_c + c], idx_r.at[lax.rem(c, IR)],
                         isem.at[lax.rem(c, IR)])

    def _idx_wait(c):
        pltpu.make_async_copy(idx_hbm.at[base_c + c], idx_r.at[lax.rem(c, IR)],
                              isem.at[lax.rem(c, IR)]).wait()

    def _gather(c, p):
        pltpu.async_copy(z_hbm.at[idx_r.at[lax.rem(c, IR), 0]], rows_v.at[p],
                         gsem.at[p])

    def _gather_wait(c, p):
        pltpu.make_async_copy(z_hbm.at[idx_r.at[lax.rem(c, IR), 0]],
                              rows_v.at[p], gsem.at[p]).wait()

    def _scatter(c, p):
        pltpu.async_copy(rows_v.at[p], acc_sh.at[idx_r.at[lax.rem(c, IR), 1]],
                         ssem.at[p], add=True)

    def _scatter_wait(c, p):
        pltpu.make_async_copy(rows_v.at[p],
                              acc_sh.at[idx_r.at[lax.rem(c, IR), 1]],
                              ssem.at[p]).wait()

    _idx_load(jnp.int32(0))
    _idx_load(jnp.int32(1))
    _idx_wait(jnp.int32(0))
    _gather(jnp.int32(0), 0)

    def _body(j, _):
        p = lax.rem(j, RB)

        @pl.when(j > 0)
        def _():
            _scatter_wait(j - 1, 1 - p)

        @pl.when(j + 2 < CH)
        def _():
            _idx_load(j + 2)

        @pl.when(j + 1 < CH)
        def _():
            _idx_wait(j + 1)
            _gather(j + 1, 1 - p)

        _gather_wait(j, p)
        _scatter(j, p)
        return _
    lax.fori_loop(0, CH, _body, None)
    _scatter_wait(jnp.int32(CH - 1), (CH - 1) % RB)
    plsc.subcore_barrier()

    nr = NP // NS
    pltpu.sync_copy(acc_sh.at[pl.ds(row0, nr), :],
                    out_hbm.at[cid, pl.ds(row0, nr), :])


def _final_body(p_ref, d_ref, bc_ref, wl_ref, bl_ref, o_ref, acc_ref):
    i = pl.program_id(0)
    nb = pl.num_programs(0)

    @pl.when(i == 0)
    def _():
        acc_ref[...] = jnp.zeros_like(acc_ref)

    blk = p_ref.shape[1]
    p = p_ref[0] + p_ref[1]
    d = d_ref[...]
    s = lax.rsqrt(jnp.maximum(d[:, 1:2] + d[:, 3:4], 1.0))
    h = jnp.maximum(p * s + bc_ref[...], 0.0)
    rows = i * blk + lax.broadcasted_iota(jnp.int32, (blk, 1), 0)
    h = jnp.where(rows < N_NODES, h, 0.0)
    acc_ref[...] += jnp.sum(h, axis=0, keepdims=True)

    @pl.when(i == nb - 1)
    def _():
        m = acc_ref[...] / float(N_NODES)
        logits = lax.dot_general(m, wl_ref[...], (((1,), (1,)), ((), ())),
                                 preferred_element_type=jnp.float32)
        logits = logits + bl_ref[...]
        e = jnp.exp(logits - jnp.max(logits))
        o_ref[...] = e / jnp.sum(e)


def kernel(in_feat, edge_index, W_conv, b_conv, W_lin, b_lin):
    src = edge_index[0].astype(jnp.int32)
    dst = edge_index[1].astype(jnp.int32)

    deg = _deg_kernel(src, dst)
    deg_t = deg.T

    z = pl.pallas_call(
        _scale_mm_body,
        grid=(25,),
        in_specs=[
            pl.BlockSpec((400, F), lambda i: (i, 0)),
            pl.BlockSpec((400, 4), lambda i: (i, 0)),
            pl.BlockSpec((F, F), lambda i: (0, 0)),
        ],
        out_specs=pl.BlockSpec((400, F), lambda i: (i, 0)),
        out_shape=jax.ShapeDtypeStruct((N_NODES, F), jnp.float32),
    )(in_feat, deg_t[:N_NODES], W_conv)

    pad = E_PAD - N_EDGES
    src_p = jnp.concatenate([src, jnp.zeros((pad,), jnp.int32)])
    dst_p = jnp.concatenate([dst, jnp.full((pad,), N_NODES, jnp.int32)])
    idx_cat = jnp.stack([src_p.reshape(-1, KE), dst_p.reshape(-1, KE)],
                        axis=1)
    partials = _edge_kernel(z, idx_cat)

    BLK = 512
    p = pl.pallas_call(
        _final_body,
        grid=(NP // BLK,),
        in_specs=[
            pl.BlockSpec((NC, BLK, F), lambda i: (0, i, 0)),
            pl.BlockSpec((BLK, 4), lambda i: (i, 0)),
            pl.BlockSpec((1, F), lambda i: (0, 0)),
            pl.BlockSpec(W_lin.shape, lambda i: (0, 0)),
            pl.BlockSpec((1, W_lin.shape[0]), lambda i: (0, 0)),
        ],
        out_specs=pl.BlockSpec((1, W_lin.shape[0]), lambda i: (0, 0)),
        out_shape=jax.ShapeDtypeStruct((1, W_lin.shape[0]), jnp.float32),
        scratch_shapes=[pltpu.VMEM((1, F), jnp.float32)],
    )(partials, deg_t, b_conv.reshape(1, F), W_lin,
      b_lin.reshape(1, W_lin.shape[0]))

    return p.reshape(W_lin.shape[0])


# speedup vs baseline: 4.8327x; 1.0008x over previous
import functools

import jax
import jax.numpy as jnp
from jax import lax
from jax.experimental import pallas as pl
from jax.experimental.pallas import tpu as pltpu
from jax.experimental.pallas import tpu_sc as plsc

NC = 2
NS = 16
L = 16
NW = NC * NS

N_NODES = 10000
N_EDGES = 320000
NP = 10240
F = 128
K = 80
EW = N_EDGES // NW
KE = 128
CH = 79
E_PAD = NW * CH * KE
IR = 4
RB = 2

_MESH = plsc.VectorSubcoreMesh(
    core_axis_name="c", subcore_axis_name="s", num_cores=NC, num_subcores=NS)


@functools.partial(
    pl.kernel,
    out_type=jax.ShapeDtypeStruct((4, NP), jnp.float32),
    mesh=_MESH,
    scratch_types=[
        pltpu.VMEM((EW,), jnp.int32),
        pltpu.VMEM((NP,), jnp.float32),
        pltpu.VMEM((NP,), jnp.float32),
        pltpu.VMEM((NS, 1, NP // NS), jnp.float32),
        pltpu.VMEM((NP // NS,), jnp.float32),
        pltpu.VMEM_SHARED((NS, 2, 1, NP), jnp.float32),
    ],
    compiler_params=pltpu.CompilerParams(needs_layout_passes=False),
)
def _deg_kernel(src_hbm, dst_hbm, out_hbm, idx_v, h_out, h_in, rbuf, rres,
                stage):
    cid = lax.axis_index("c")
    sid = lax.axis_index("s")
    wid = sid * NC + cid
    base = wid * EW
    ones = jnp.ones((L,), jnp.float32)
    zeros = jnp.zeros((L,), jnp.float32)

    def _zero(i, _):
        h_out[pl.ds(i * L, L)] = zeros
        h_in[pl.ds(i * L, L)] = zeros
        return _
    lax.fori_loop(0, NP // L, _zero, None)

    pltpu.sync_copy(src_hbm.at[pl.ds(base, EW)], idx_v)
    def _hist_out(i, _):
        plsc.addupdate_scatter(h_out, [idx_v[pl.ds(i * L, L)]], ones)
        return _
    lax.fori_loop(0, EW // L, _hist_out, None)

    pltpu.sync_copy(dst_hbm.at[pl.ds(base, EW)], idx_v)
    def _hist_in(i, _):
        plsc.addupdate_scatter(h_in, [idx_v[pl.ds(i * L, L)]], ones)
        return _
    lax.fori_loop(0, EW // L, _hist_in, None)

    pltpu.sync_copy(h_out, stage.at[sid, 0, 0, :])
    pltpu.sync_copy(h_in, stage.at[sid, 1, 0, :])
    plsc.subcore_barrier()

    nc_ = NP // NS
    col0 = sid * nc_
    for r in range(2):
        def _pull(t, _):
            pltpu.sync_copy(stage.at[t, r, 0, pl.ds(col0, nc_)], rbuf.at[t, 0])
            return _
        lax.fori_loop(0, NS, _pull, None)
        def _red(i, _):
            acc = rbuf[0, 0, pl.ds(i * L, L)]
            for t in range(1, NS):
                acc = acc + rbuf[t, 0, pl.ds(i * L, L)]
            rres[pl.ds(i * L, L)] = acc
            return _
        lax.fori_loop(0, nc_ // L, _red, None)
        pltpu.sync_copy(rres, out_hbm.at[2 * cid + r, pl.ds(col0, nc_)])


def _scale_mm_body(x_ref, d_ref, w_ref, o_ref):
    d = d_ref[...]
    s = lax.rsqrt(jnp.maximum(d[:, 0:1] + d[:, 2:3], 1.0))
    o_ref[...] = jnp.dot(x_ref[...] * s, w_ref[...],
                         preferred_element_type=jnp.float32)


@functools.partial(
    pl.kernel,
    out_type=jax.ShapeDtypeStruct((NC, NP, F), jnp.float32),
    mesh=_MESH,
    scratch_types=[
        pltpu.VMEM((IR, 2, KE), jnp.int32),
        pltpu.VMEM((RB, KE, F), jnp.float32),
        pltpu.VMEM_SHARED((NP, F), jnp.float32),
        pltpu.SemaphoreType.DMA((IR,)),
        pltpu.SemaphoreType.DMA((RB,)),
        pltpu.SemaphoreType.DMA((RB,)),
    ],
)
def _edge_kernel(z_hbm, idx_hbm, out_hbm, idx_r, rows_v, acc_sh,
                 isem, gsem, ssem):
    cid = lax.axis_index("c")
    sid = lax.axis_index("s")
    wid = sid * NC + cid
    base_c = wid * CH

    def _fill(i, _):
        r = i // (F // L)
        c = lax.rem(i, F // L)
        rows_v[0, r, pl.ds(c * L, L)] = jnp.zeros((L,), jnp.float32)
        return _
    lax.fori_loop(0, KE * (F // L), _fill, None)
    row0 = sid * (NP // NS)
    def _zero(k, _):
        pltpu.sync_copy(rows_v.at[0], acc_sh.at[pl.ds(row0 + k * KE, KE), :])
        return _
    lax.fori_loop(0, (NP // NS) // KE, _zero, None)
    plsc.subcore_barrier()

    def _idx_load(c):
        pltpu.async_copy(idx_hbm.at[base_c + c], idx_r.at[lax.rem(c, IR)],
                         isem.at[lax.rem(c, IR)])

    def _idx_wait(c):
        pltpu.make_async_copy(idx_hbm.at[base_c + c], idx_r.at[lax.rem(c, IR)],
                              isem.at[lax.rem(c, IR)]).wait()

    def _gather(c, p):
        pltpu.async_copy(z_hbm.at[idx_r.at[lax.rem(c, IR), 0]], rows_v.at[p],
                         gsem.at[p])

    def _gather_wait(c, p):
        pltpu.make_async_copy(z_hbm.at[idx_r.at[lax.rem(c, IR), 0]],
                              rows_v.at[p], gsem.at[p]).wait()

    def _scatter(c, p):
        pltpu.async_copy(rows_v.at[p], acc_sh.at[idx_r.at[lax.rem(c, IR), 1]],
                         ssem.at[p], add=True)

    def _scatter_wait(c, p):
        pltpu.make_async_copy(rows_v.at[p],
                              acc_sh.at[idx_r.at[lax.rem(c, IR), 1]],
                              ssem.at[p]).wait()

    _idx_load(jnp.int32(0))
    _idx_load(jnp.int32(1))
    _idx_wait(jnp.int32(0))
    _gather(jnp.int32(0), 0)

    def _body(j, _):
        p = lax.rem(j, RB)

        @pl.when(j > 0)
        def _():
            _scatter_wait(j - 1, 1 - p)

        @pl.when(j + 2 < CH)
        def _():
            _idx_load(j + 2)

        @pl.when(j + 1 < CH)
        def _():
            _idx_wait(j + 1)
            _gather(j + 1, 1 - p)

        _gather_wait(j, p)
        _scatter(j, p)
        return _
    lax.fori_loop(0, CH, _body, None)
    _scatter_wait(jnp.int32(CH - 1), (CH - 1) % RB)
    plsc.subcore_barrier()

    nr = NP // NS
    pltpu.sync_copy(acc_sh.at[pl.ds(row0, nr), :],
                    out_hbm.at[cid, pl.ds(row0, nr), :])


def _final_body(p_ref, d_ref, bc_ref, wl_ref, bl_ref, o_ref, acc_ref):
    i = pl.program_id(0)
    nb = pl.num_programs(0)

    @pl.when(i == 0)
    def _():
        acc_ref[...] = jnp.zeros_like(acc_ref)

    blk = p_ref.shape[1]
    p = p_ref[0] + p_ref[1]
    d = d_ref[...]
    s = lax.rsqrt(jnp.maximum(d[:, 1:2] + d[:, 3:4], 1.0))
    h = jnp.maximum(p * s + bc_ref[...], 0.0)
    rows = i * blk + lax.broadcasted_iota(jnp.int32, (blk, 1), 0)
    h = jnp.where(rows < N_NODES, h, 0.0)
    acc_ref[...] += jnp.sum(h, axis=0, keepdims=True)

    @pl.when(i == nb - 1)
    def _():
        m = acc_ref[...] / float(N_NODES)
        logits = lax.dot_general(m, wl_ref[...], (((1,), (1,)), ((), ())),
                                 preferred_element_type=jnp.float32)
        logits = logits + bl_ref[...]
        e = jnp.exp(logits - jnp.max(logits))
        o_ref[...] = e / jnp.sum(e)


def kernel(in_feat, edge_index, W_conv, b_conv, W_lin, b_lin):
    src = edge_index[0].astype(jnp.int32)
    dst = edge_index[1].astype(jnp.int32)

    deg = _deg_kernel(src, dst)
    deg_t = deg.T

    z = pl.pallas_call(
        _scale_mm_body,
        grid=(25,),
        in_specs=[
            pl.BlockSpec((400, F), lambda i: (i, 0)),
            pl.BlockSpec((400, 4), lambda i: (i, 0)),
            pl.BlockSpec((F, F), lambda i: (0, 0)),
        ],
        out_specs=pl.BlockSpec((400, F), lambda i: (i, 0)),
        out_shape=jax.ShapeDtypeStruct((N_NODES, F), jnp.float32),
    )(in_feat, deg_t[:N_NODES], W_conv)

    pad = E_PAD - N_EDGES
    src_p = jnp.concatenate([src, jnp.zeros((pad,), jnp.int32)])
    dst_p = jnp.concatenate(
        [dst, N_NODES + jnp.arange(pad, dtype=jnp.int32) % (NP - N_NODES)])
    idx_cat = jnp.stack([src_p.reshape(-1, KE), dst_p.reshape(-1, KE)],
                        axis=1)
    partials = _edge_kernel(z, idx_cat)

    BLK = 512
    p = pl.pallas_call(
        _final_body,
        grid=(NP // BLK,),
        in_specs=[
            pl.BlockSpec((NC, BLK, F), lambda i: (0, i, 0)),
            pl.BlockSpec((BLK, 4), lambda i: (i, 0)),
            pl.BlockSpec((1, F), lambda i: (0, 0)),
            pl.BlockSpec(W_lin.shape, lambda i: (0, 0)),
            pl.BlockSpec((1, W_lin.shape[0]), lambda i: (0, 0)),
        ],
        out_specs=pl.BlockSpec((1, W_lin.shape[0]), lambda i: (0, 0)),
        out_shape=jax.ShapeDtypeStruct((1, W_lin.shape[0]), jnp.float32),
        scratch_shapes=[pltpu.VMEM((1, F), jnp.float32)],
    )(partials, deg_t, b_conv.reshape(1, F), W_lin,
      b_lin.reshape(1, W_lin.shape[0]))

    return p.reshape(W_lin.shape[0])


# speedup vs baseline: 5.1027x; 1.0559x over previous
import functools

import jax
import jax.numpy as jnp
from jax import lax
from jax.experimental import pallas as pl
from jax.experimental.pallas import tpu as pltpu
from jax.experimental.pallas import tpu_sc as plsc

NC = 2
NS = 16
L = 16
NW = NC * NS

N_NODES = 10000
N_EDGES = 320000
NP = 10240
F = 128
K = 80
EW = N_EDGES // NW
KE = 128
CH = 79
CH0 = 111
CH1 = 2 * CH - CH0
E_PAD = NW * CH * KE
IR = 4
RB = 2

_MESH = plsc.VectorSubcoreMesh(
    core_axis_name="c", subcore_axis_name="s", num_cores=NC, num_subcores=NS)


@functools.partial(
    pl.kernel,
    out_type=jax.ShapeDtypeStruct((4, NP), jnp.float32),
    mesh=_MESH,
    scratch_types=[
        pltpu.VMEM((EW,), jnp.int32),
        pltpu.VMEM((NP,), jnp.float32),
        pltpu.VMEM((NP,), jnp.float32),
        pltpu.VMEM((NS, 1, NP // NS), jnp.float32),
        pltpu.VMEM((NP // NS,), jnp.float32),
        pltpu.VMEM_SHARED((NS, 2, 1, NP), jnp.float32),
    ],
    compiler_params=pltpu.CompilerParams(needs_layout_passes=False),
)
def _deg_kernel(src_hbm, dst_hbm, out_hbm, idx_v, h_out, h_in, rbuf, rres,
                stage):
    cid = lax.axis_index("c")
    sid = lax.axis_index("s")
    wid = sid * NC + cid
    base = wid * EW
    ones = jnp.ones((L,), jnp.float32)
    zeros = jnp.zeros((L,), jnp.float32)

    def _zero(i, _):
        h_out[pl.ds(i * L, L)] = zeros
        h_in[pl.ds(i * L, L)] = zeros
        return _
    lax.fori_loop(0, NP // L, _zero, None)

    pltpu.sync_copy(src_hbm.at[pl.ds(base, EW)], idx_v)
    def _hist_out(i, _):
        plsc.addupdate_scatter(h_out, [idx_v[pl.ds(i * L, L)]], ones)
        return _
    lax.fori_loop(0, EW // L, _hist_out, None)

    pltpu.sync_copy(dst_hbm.at[pl.ds(base, EW)], idx_v)
    def _hist_in(i, _):
        plsc.addupdate_scatter(h_in, [idx_v[pl.ds(i * L, L)]], ones)
        return _
    lax.fori_loop(0, EW // L, _hist_in, None)

    pltpu.sync_copy(h_out, stage.at[sid, 0, 0, :])
    pltpu.sync_copy(h_in, stage.at[sid, 1, 0, :])
    plsc.subcore_barrier()

    nc_ = NP // NS
    col0 = sid * nc_
    for r in range(2):
        def _pull(t, _):
            pltpu.sync_copy(stage.at[t, r, 0, pl.ds(col0, nc_)], rbuf.at[t, 0])
            return _
        lax.fori_loop(0, NS, _pull, None)
        def _red(i, _):
            acc = rbuf[0, 0, pl.ds(i * L, L)]
            for t in range(1, NS):
                acc = acc + rbuf[t, 0, pl.ds(i * L, L)]
            rres[pl.ds(i * L, L)] = acc
            return _
        lax.fori_loop(0, nc_ // L, _red, None)
        pltpu.sync_copy(rres, out_hbm.at[2 * cid + r, pl.ds(col0, nc_)])


def _scale_mm_body(x_ref, d_ref, w_ref, o_ref):
    d = d_ref[...]
    s = lax.rsqrt(jnp.maximum(d[:, 0:1] + d[:, 2:3], 1.0))
    o_ref[...] = jnp.dot(x_ref[...] * s, w_ref[...],
                         preferred_element_type=jnp.float32)


@functools.partial(
    pl.kernel,
    out_type=jax.ShapeDtypeStruct((NC, NP, F), jnp.float32),
    mesh=_MESH,
    scratch_types=[
        pltpu.VMEM((IR, 2, KE), jnp.int32),
        pltpu.VMEM((RB, KE, F), jnp.float32),
        pltpu.VMEM_SHARED((NP, F), jnp.float32),
        pltpu.SemaphoreType.DMA((IR,)),
        pltpu.SemaphoreType.DMA((RB,)),
        pltpu.SemaphoreType.DMA((RB,)),
    ],
)
def _edge_kernel(z_hbm, idx_hbm, out_hbm, idx_r, rows_v, acc_sh,
                 isem, gsem, ssem):
    cid = lax.axis_index("c")
    sid = lax.axis_index("s")
    base_c = jnp.where(cid == 0, sid * CH0, NS * CH0 + sid * CH1)
    nch = jnp.where(cid == 0, CH0, CH1)

    def _fill(i, _):
        r = i // (F // L)
        c = lax.rem(i, F // L)
        rows_v[0, r, pl.ds(c * L, L)] = jnp.zeros((L,), jnp.float32)
        return _
    lax.fori_loop(0, KE * (F // L), _fill, None)
    row0 = sid * (NP // NS)
    def _zero(k, _):
        pltpu.sync_copy(rows_v.at[0], acc_sh.at[pl.ds(row0 + k * KE, KE), :])
        return _
    lax.fori_loop(0, (NP // NS) // KE, _zero, None)
    plsc.subcore_barrier()

    def _idx_load(c):
        pltpu.async_copy(idx_hbm.at[base_c + c], idx_r.at[lax.rem(c, IR)],
                         isem.at[lax.rem(c, IR)])

    def _idx_wait(c):
        pltpu.make_async_copy(idx_hbm.at[base_c + c], idx_r.at[lax.rem(c, IR)],
                              isem.at[lax.rem(c, IR)]).wait()

    def _gather(c, p):
        pltpu.async_copy(z_hbm.at[idx_r.at[lax.rem(c, IR), 0]], rows_v.at[p],
                         gsem.at[p])

    def _gather_wait(c, p):
        pltpu.make_async_copy(z_hbm.at[idx_r.at[lax.rem(c, IR), 0]],
                              rows_v.at[p], gsem.at[p]).wait()

    def _scatter(c, p):
        pltpu.async_copy(rows_v.at[p], acc_sh.at[idx_r.at[lax.rem(c, IR), 1]],
                         ssem.at[p], add=True)

    def _scatter_wait(c, p):
        pltpu.make_async_copy(rows_v.at[p],
                              acc_sh.at[idx_r.at[lax.rem(c, IR), 1]],
                              ssem.at[p]).wait()

    _idx_load(jnp.int32(0))
    _idx_load(jnp.int32(1))
    _idx_wait(jnp.int32(0))
    _gather(jnp.int32(0), 0)

    def _body(j, _):
        p = lax.rem(j, RB)

        @pl.when(j > 0)
        def _():
            _scatter_wait(j - 1, 1 - p)

        @pl.when(j + 2 < nch)
        def _():
            _idx_load(j + 2)

        @pl.when(j + 1 < nch)
        def _():
            _idx_wait(j + 1)
            _gather(j + 1, 1 - p)

        _gather_wait(j, p)
        _scatter(j, p)
        return _
    lax.fori_loop(0, nch, _body, None)
    _scatter_wait(nch - 1, lax.rem(nch - 1, RB))
    plsc.subcore_barrier()

    nr = NP // NS
    pltpu.sync_copy(acc_sh.at[pl.ds(row0, nr), :],
                    out_hbm.at[cid, pl.ds(row0, nr), :])


def _final_body(p_ref, d_ref, bc_ref, wl_ref, bl_ref, o_ref, acc_ref):
    i = pl.program_id(0)
    nb = pl.num_programs(0)

    @pl.when(i == 0)
    def _():
        acc_ref[...] = jnp.zeros_like(acc_ref)

    blk = p_ref.shape[1]
    p = p_ref[0] + p_ref[1]
    d = d_ref[...]
    s = lax.rsqrt(jnp.maximum(d[:, 1:2] + d[:, 3:4], 1.0))
    h = jnp.maximum(p * s + bc_ref[...], 0.0)
    rows = i * blk + lax.broadcasted_iota(jnp.int32, (blk, 1), 0)
    h = jnp.where(rows < N_NODES, h, 0.0)
    acc_ref[...] += jnp.sum(h, axis=0, keepdims=True)

    @pl.when(i == nb - 1)
    def _():
        m = acc_ref[...] / float(N_NODES)
        logits = lax.dot_general(m, wl_ref[...], (((1,), (1,)), ((), ())),
                                 preferred_element_type=jnp.float32)
        logits = logits + bl_ref[...]
        e = jnp.exp(logits - jnp.max(logits))
        o_ref[...] = e / jnp.sum(e)


def kernel(in_feat, edge_index, W_conv, b_conv, W_lin, b_lin):
    src = edge_index[0].astype(jnp.int32)
    dst = edge_index[1].astype(jnp.int32)

    deg = _deg_kernel(src, dst)
    deg_t = deg.T

    z = pl.pallas_call(
        _scale_mm_body,
        grid=(25,),
        in_specs=[
            pl.BlockSpec((400, F), lambda i: (i, 0)),
            pl.BlockSpec((400, 4), lambda i: (i, 0)),
            pl.BlockSpec((F, F), lambda i: (0, 0)),
        ],
        out_specs=pl.BlockSpec((400, F), lambda i: (i, 0)),
        out_shape=jax.ShapeDtypeStruct((N_NODES, F), jnp.float32),
    )(in_feat, deg_t[:N_NODES], W_conv)

    pad = E_PAD - N_EDGES
    src_p = jnp.concatenate([src, jnp.zeros((pad,), jnp.int32)])
    dst_p = jnp.concatenate(
        [dst, N_NODES + jnp.arange(pad, dtype=jnp.int32) % (NP - N_NODES)])
    idx_cat = jnp.stack([src_p.reshape(-1, KE), dst_p.reshape(-1, KE)],
                        axis=1)
    partials = _edge_kernel(z, idx_cat)

    BLK = 512
    p = pl.pallas_call(
        _final_body,
        grid=(NP // BLK,),
        in_specs=[
            pl.BlockSpec((NC, BLK, F), lambda i: (0, i, 0)),
            pl.BlockSpec((BLK, 4), lambda i: (i, 0)),
            pl.BlockSpec((1, F), lambda i: (0, 0)),
            pl.BlockSpec(W_lin.shape, lambda i: (0, 0)),
            pl.BlockSpec((1, W_lin.shape[0]), lambda i: (0, 0)),
        ],
        out_specs=pl.BlockSpec((1, W_lin.shape[0]), lambda i: (0, 0)),
        out_shape=jax.ShapeDtypeStruct((1, W_lin.shape[0]), jnp.float32),
        scratch_shapes=[pltpu.VMEM((1, F), jnp.float32)],
    )(partials, deg_t, b_conv.reshape(1, F), W_lin,
      b_lin.reshape(1, W_lin.shape[0]))

    return p.reshape(W_lin.shape[0])


# speedup vs baseline: 8.4131x; 1.6487x over previous
import functools

import jax
import jax.numpy as jnp
from jax import lax
from jax.experimental import pallas as pl
from jax.experimental.pallas import tpu as pltpu
from jax.experimental.pallas import tpu_sc as plsc

NC = 2
NS = 16
L = 16
NW = NC * NS

N_NODES = 10000
N_EDGES = 320000
NP = 10240
F = 128
K = 80
EW = N_EDGES // NW
KE = 80
CH = 125
CH0 = 170
CH1 = 2 * CH - CH0
IR = 6
RB = 4

_MESH = plsc.VectorSubcoreMesh(
    core_axis_name="c", subcore_axis_name="s", num_cores=NC, num_subcores=NS)


@functools.partial(
    pl.kernel,
    out_type=jax.ShapeDtypeStruct((4, NP), jnp.float32),
    mesh=_MESH,
    scratch_types=[
        pltpu.VMEM((EW,), jnp.int32),
        pltpu.VMEM((NP,), jnp.float32),
        pltpu.VMEM((NP,), jnp.float32),
        pltpu.VMEM((NS, 1, NP // NS), jnp.float32),
        pltpu.VMEM((NP // NS,), jnp.float32),
        pltpu.VMEM_SHARED((NS, 2, 1, NP), jnp.float32),
    ],
    compiler_params=pltpu.CompilerParams(needs_layout_passes=False),
)
def _deg_kernel(src_hbm, dst_hbm, out_hbm, idx_v, h_out, h_in, rbuf, rres,
                stage):
    cid = lax.axis_index("c")
    sid = lax.axis_index("s")
    wid = sid * NC + cid
    base = wid * EW
    ones = jnp.ones((L,), jnp.float32)
    zeros = jnp.zeros((L,), jnp.float32)

    def _zero(i, _):
        h_out[pl.ds(i * L, L)] = zeros
        h_in[pl.ds(i * L, L)] = zeros
        return _
    lax.fori_loop(0, NP // L, _zero, None)

    pltpu.sync_copy(src_hbm.at[pl.ds(base, EW)], idx_v)
    def _hist_out(i, _):
        plsc.addupdate_scatter(h_out, [idx_v[pl.ds(i * L, L)]], ones)
        return _
    lax.fori_loop(0, EW // L, _hist_out, None)

    pltpu.sync_copy(dst_hbm.at[pl.ds(base, EW)], idx_v)
    def _hist_in(i, _):
        plsc.addupdate_scatter(h_in, [idx_v[pl.ds(i * L, L)]], ones)
        return _
    lax.fori_loop(0, EW // L, _hist_in, None)

    pltpu.sync_copy(h_out, stage.at[sid, 0, 0, :])
    pltpu.sync_copy(h_in, stage.at[sid, 1, 0, :])
    plsc.subcore_barrier()

    nc_ = NP // NS
    col0 = sid * nc_
    for r in range(2):
        def _pull(t, _):
            pltpu.sync_copy(stage.at[t, r, 0, pl.ds(col0, nc_)], rbuf.at[t, 0])
            return _
        lax.fori_loop(0, NS, _pull, None)
        def _red(i, _):
            acc = rbuf[0, 0, pl.ds(i * L, L)]
            for t in range(1, NS):
                acc = acc + rbuf[t, 0, pl.ds(i * L, L)]
            rres[pl.ds(i * L, L)] = acc
            return _
        lax.fori_loop(0, nc_ // L, _red, None)
        pltpu.sync_copy(rres, out_hbm.at[2 * cid + r, pl.ds(col0, nc_)])


def _scale_mm_body(x_ref, d_ref, w_ref, o_ref):
    d = d_ref[...]
    s = lax.rsqrt(jnp.maximum(d[:, 0:1] + d[:, 2:3], 1.0))
    o_ref[...] = jnp.dot(x_ref[...] * s, w_ref[...],
                         preferred_element_type=jnp.float32)


@functools.partial(
    pl.kernel,
    out_type=jax.ShapeDtypeStruct((NC, NP, F), jnp.float32),
    mesh=_MESH,
    scratch_types=[
        pltpu.VMEM((IR, 2, KE), jnp.int32),
        pltpu.VMEM((RB, KE, F), jnp.float32),
        pltpu.VMEM_SHARED((NP, F), jnp.float32),
        pltpu.SemaphoreType.DMA((IR,)),
        pltpu.SemaphoreType.DMA((RB,)),
        pltpu.SemaphoreType.DMA((RB,)),
    ],
)
def _edge_kernel(z_hbm, idx_hbm, out_hbm, idx_r, rows_v, acc_sh,
                 isem, gsem, ssem):
    cid = lax.axis_index("c")
    sid = lax.axis_index("s")
    base_c = jnp.where(cid == 0, sid * CH0, NS * CH0 + sid * CH1)
    nch = jnp.where(cid == 0, CH0, CH1)

    def _fill(i, _):
        r = i // (F // L)
        c = lax.rem(i, F // L)
        rows_v[0, r, pl.ds(c * L, L)] = jnp.zeros((L,), jnp.float32)
        return _
    lax.fori_loop(0, KE * (F // L), _fill, None)
    row0 = sid * (NP // NS)
    def _zero(k, _):
        pltpu.sync_copy(rows_v.at[0], acc_sh.at[pl.ds(row0 + k * KE, KE), :])
        return _
    lax.fori_loop(0, (NP // NS) // KE, _zero, None)
    plsc.subcore_barrier()

    def _idx_load(c):
        pltpu.async_copy(idx_hbm.at[base_c + c], idx_r.at[lax.rem(c, IR)],
                         isem.at[lax.rem(c, IR)])

    def _idx_wait(c):
        pltpu.make_async_copy(idx_hbm.at[base_c + c], idx_r.at[lax.rem(c, IR)],
                              isem.at[lax.rem(c, IR)]).wait()

    def _gather(c, p):
        pltpu.async_copy(z_hbm.at[idx_r.at[lax.rem(c, IR), 0]], rows_v.at[p],
                         gsem.at[p])

    def _gather_wait(c, p):
        pltpu.make_async_copy(z_hbm.at[idx_r.at[lax.rem(c, IR), 0]],
                              rows_v.at[p], gsem.at[p]).wait()

    def _scatter(c, p):
        pltpu.async_copy(rows_v.at[p], acc_sh.at[idx_r.at[lax.rem(c, IR), 1]],
                         ssem.at[p], add=True)

    def _scatter_wait(c, p):
        pltpu.make_async_copy(rows_v.at[p],
                              acc_sh.at[idx_r.at[lax.rem(c, IR), 1]],
                              ssem.at[p]).wait()

    for c in range(IR - 1):
        _idx_load(jnp.int32(c))
    for c in range(RB - 1):
        _idx_wait(jnp.int32(c))
        _gather(jnp.int32(c), c)

    def _body(j, _):
        p = lax.rem(j, RB)

        @pl.when(j > 0)
        def _():
            _scatter_wait(j - 1, lax.rem(j - 1, RB))

        @pl.when(j + IR - 1 < nch)
        def _():
            _idx_load(j + IR - 1)

        @pl.when(j + RB - 1 < nch)
        def _():
            _idx_wait(j + RB - 1)
            _gather(j + RB - 1, lax.rem(j + RB - 1, RB))

        _gather_wait(j, p)
        _scatter(j, p)
        return _
    lax.fori_loop(0, nch, _body, None)
    _scatter_wait(nch - 1, lax.rem(nch - 1, RB))
    plsc.subcore_barrier()

    nr = NP // NS
    pltpu.sync_copy(acc_sh.at[pl.ds(row0, nr), :],
                    out_hbm.at[cid, pl.ds(row0, nr), :])


def _final_body(p_ref, d_ref, bc_ref, wl_ref, bl_ref, o_ref, acc_ref):
    i = pl.program_id(0)
    nb = pl.num_programs(0)

    @pl.when(i == 0)
    def _():
        acc_ref[...] = jnp.zeros_like(acc_ref)

    blk = p_ref.shape[1]
    p = p_ref[0] + p_ref[1]
    d = d_ref[...]
    s = lax.rsqrt(jnp.maximum(d[:, 1:2] + d[:, 3:4], 1.0))
    h = jnp.maximum(p * s + bc_ref[...], 0.0)
    rows = i * blk + lax.broadcasted_iota(jnp.int32, (blk, 1), 0)
    h = jnp.where(rows < N_NODES, h, 0.0)
    acc_ref[...] += jnp.sum(h, axis=0, keepdims=True)

    @pl.when(i == nb - 1)
    def _():
        m = acc_ref[...] / float(N_NODES)
        logits = lax.dot_general(m, wl_ref[...], (((1,), (1,)), ((), ())),
                                 preferred_element_type=jnp.float32)
        logits = logits + bl_ref[...]
        e = jnp.exp(logits - jnp.max(logits))
        o_ref[...] = e / jnp.sum(e)


def kernel(in_feat, edge_index, W_conv, b_conv, W_lin, b_lin):
    src = edge_index[0].astype(jnp.int32)
    dst = edge_index[1].astype(jnp.int32)

    deg = _deg_kernel(src, dst)
    deg_t = deg.T

    z = pl.pallas_call(
        _scale_mm_body,
        grid=(25,),
        in_specs=[
            pl.BlockSpec((400, F), lambda i: (i, 0)),
            pl.BlockSpec((400, 4), lambda i: (i, 0)),
            pl.BlockSpec((F, F), lambda i: (0, 0)),
        ],
        out_specs=pl.BlockSpec((400, F), lambda i: (i, 0)),
        out_shape=jax.ShapeDtypeStruct((N_NODES, F), jnp.float32),
    )(in_feat, deg_t[:N_NODES], W_conv)

    idx_cat = jnp.stack([src.reshape(-1, KE), dst.reshape(-1, KE)],
                        axis=1)
    partials = _edge_kernel(z, idx_cat)

    BLK = 512
    p = pl.pallas_call(
        _final_body,
        grid=(NP // BLK,),
        in_specs=[
            pl.BlockSpec((NC, BLK, F), lambda i: (0, i, 0)),
            pl.BlockSpec((BLK, 4), lambda i: (i, 0)),
            pl.BlockSpec((1, F), lambda i: (0, 0)),
            pl.BlockSpec(W_lin.shape, lambda i: (0, 0)),
            pl.BlockSpec((1, W_lin.shape[0]), lambda i: (0, 0)),
        ],
        out_specs=pl.BlockSpec((1, W_lin.shape[0]), lambda i: (0, 0)),
        out_shape=jax.ShapeDtypeStruct((1, W_lin.shape[0]), jnp.float32),
        scratch_shapes=[pltpu.VMEM((1, F), jnp.float32)],
    )(partials, deg_t, b_conv.reshape(1, F), W_lin,
      b_lin.reshape(1, W_lin.shape[0]))

    return p.reshape(W_lin.shape[0])


# speedup vs baseline: 10.5873x; 1.2584x over previous
import functools

import jax
import jax.numpy as jnp
from jax import lax
from jax.experimental import pallas as pl
from jax.experimental.pallas import tpu as pltpu
from jax.experimental.pallas import tpu_sc as plsc

NC = 2
NS = 16
L = 16
NW = NC * NS

N_NODES = 10000
N_EDGES = 320000
NP = 10240
F = 128
K = 80
EW = N_EDGES // NW
KE = 80
CH = 125
CH0 = 134
CH1 = 2 * CH - CH0
IR = 6
RB = 4

_MESH = plsc.VectorSubcoreMesh(
    core_axis_name="c", subcore_axis_name="s", num_cores=NC, num_subcores=NS)


@functools.partial(
    pl.kernel,
    out_type=jax.ShapeDtypeStruct((4, NP), jnp.float32),
    mesh=_MESH,
    scratch_types=[
        pltpu.VMEM((EW,), jnp.int32),
        pltpu.VMEM((NP,), jnp.float32),
        pltpu.VMEM((NP,), jnp.float32),
        pltpu.VMEM((NS, 1, NP // NS), jnp.float32),
        pltpu.VMEM((NP // NS,), jnp.float32),
        pltpu.VMEM_SHARED((NS, 2, 1, NP), jnp.float32),
    ],
    compiler_params=pltpu.CompilerParams(needs_layout_passes=False),
)
def _deg_kernel(ei_hbm, out_hbm, idx_v, h_out, h_in, rbuf, rres,
                stage):
    cid = lax.axis_index("c")
    sid = lax.axis_index("s")
    wid = sid * NC + cid
    base = wid * EW
    ones = jnp.ones((L,), jnp.float32)
    zeros = jnp.zeros((L,), jnp.float32)

    def _zero(i, _):
        h_out[pl.ds(i * L, L)] = zeros
        h_in[pl.ds(i * L, L)] = zeros
        return _
    lax.fori_loop(0, NP // L, _zero, None)

    pltpu.sync_copy(ei_hbm.at[0, wid, 0, :], idx_v)
    def _hist_out(i, _):
        for k in range(5):
            plsc.addupdate_scatter(
                h_out, [idx_v[pl.ds((i * 5 + k) * L, L)]], ones)
        return _
    lax.fori_loop(0, EW // (5 * L), _hist_out, None)

    pltpu.sync_copy(ei_hbm.at[1, wid, 0, :], idx_v)
    def _hist_in(i, _):
        for k in range(5):
            plsc.addupdate_scatter(
                h_in, [idx_v[pl.ds((i * 5 + k) * L, L)]], ones)
        return _
    lax.fori_loop(0, EW // (5 * L), _hist_in, None)

    pltpu.sync_copy(h_out, stage.at[sid, 0, 0, :])
    pltpu.sync_copy(h_in, stage.at[sid, 1, 0, :])
    plsc.subcore_barrier()

    nc_ = NP // NS
    col0 = sid * nc_
    for r in range(2):
        def _pull(t, _):
            pltpu.sync_copy(stage.at[t, r, 0, pl.ds(col0, nc_)], rbuf.at[t, 0])
            return _
        lax.fori_loop(0, NS, _pull, None)
        def _red(i, _):
            acc = rbuf[0, 0, pl.ds(i * L, L)]
            for t in range(1, NS):
                acc = acc + rbuf[t, 0, pl.ds(i * L, L)]
            rres[pl.ds(i * L, L)] = acc
            return _
        lax.fori_loop(0, nc_ // L, _red, None)
        pltpu.sync_copy(rres, out_hbm.at[2 * cid + r, pl.ds(col0, nc_)])


def _scale_mm_body(x_ref, d_ref, w_ref, o_ref):
    d = d_ref[...]
    s = lax.rsqrt(jnp.maximum(d[:, 0:1] + d[:, 2:3], 1.0))
    o_ref[...] = jnp.dot(x_ref[...] * s, w_ref[...],
                         preferred_element_type=jnp.float32)


@functools.partial(
    pl.kernel,
    out_type=jax.ShapeDtypeStruct((NC, NP, F), jnp.float32),
    mesh=_MESH,
    scratch_types=[
        pltpu.VMEM((IR, 2, KE), jnp.int32),
        pltpu.VMEM((RB, KE, F), jnp.float32),
        pltpu.VMEM_SHARED((NP, F), jnp.float32),
        pltpu.SemaphoreType.DMA((IR,)),
        pltpu.SemaphoreType.DMA((RB,)),
        pltpu.SemaphoreType.DMA((RB,)),
    ],
)
def _edge_kernel(z_hbm, ei_hbm, out_hbm, idx_r, rows_v, acc_sh,
                 isem, gsem, ssem):
    cid = lax.axis_index("c")
    sid = lax.axis_index("s")
    base_c = jnp.where(cid == 0, sid * CH0, NS * CH0 + sid * CH1)
    nch = jnp.where(cid == 0, CH0, CH1)

    def _fill(i, _):
        r = i // (F // L)
        c = lax.rem(i, F // L)
        rows_v[0, r, pl.ds(c * L, L)] = jnp.zeros((L,), jnp.float32)
        return _
    lax.fori_loop(0, KE * (F // L), _fill, None)
    row0 = sid * (NP // NS)
    def _zero(k, _):
        pltpu.sync_copy(rows_v.at[0], acc_sh.at[pl.ds(row0 + k * KE, KE), :])
        return _
    lax.fori_loop(0, (NP // NS) // KE, _zero, None)
    plsc.subcore_barrier()

    def _idx_load(c):
        pltpu.async_copy(ei_hbm.at[0, base_c + c, 0], idx_r.at[lax.rem(c, IR), 0],
                         isem.at[lax.rem(c, IR)])
        pltpu.async_copy(ei_hbm.at[1, base_c + c, 0], idx_r.at[lax.rem(c, IR), 1],
                         isem.at[lax.rem(c, IR)])

    def _idx_wait(c):
        pltpu.make_async_copy(ei_hbm.at[0, base_c + c, 0],
                              idx_r.at[lax.rem(c, IR), 0],
                              isem.at[lax.rem(c, IR)]).wait()
        pltpu.make_async_copy(ei_hbm.at[1, base_c + c, 0],
                              idx_r.at[lax.rem(c, IR), 1],
                              isem.at[lax.rem(c, IR)]).wait()

    def _gather(c, p):
        pltpu.async_copy(z_hbm.at[idx_r.at[lax.rem(c, IR), 0]], rows_v.at[p],
                         gsem.at[p])

    def _gather_wait(c, p):
        pltpu.make_async_copy(z_hbm.at[idx_r.at[lax.rem(c, IR), 0]],
                              rows_v.at[p], gsem.at[p]).wait()

    def _scatter(c, p):
        pltpu.async_copy(rows_v.at[p], acc_sh.at[idx_r.at[lax.rem(c, IR), 1]],
                         ssem.at[p], add=True)

    def _scatter_wait(c, p):
        pltpu.make_async_copy(rows_v.at[p],
                              acc_sh.at[idx_r.at[lax.rem(c, IR), 1]],
                              ssem.at[p]).wait()

    for c in range(IR - 1):
        _idx_load(jnp.int32(c))
    for c in range(RB - 1):
        _idx_wait(jnp.int32(c))
        _gather(jnp.int32(c), c)

    def _body(j, _):
        p = lax.rem(j, RB)

        @pl.when(j > 0)
        def _():
            _scatter_wait(j - 1, lax.rem(j - 1, RB))

        @pl.when(j + IR - 1 < nch)
        def _():
            _idx_load(j + IR - 1)

        @pl.when(j + RB - 1 < nch)
        def _():
            _idx_wait(j + RB - 1)
            _gather(j + RB - 1, lax.rem(j + RB - 1, RB))

        _gather_wait(j, p)
        _scatter(j, p)
        return _
    lax.fori_loop(0, nch, _body, None)
    _scatter_wait(nch - 1, lax.rem(nch - 1, RB))
    plsc.subcore_barrier()

    nr = NP // NS
    pltpu.sync_copy(acc_sh.at[pl.ds(row0, nr), :],
                    out_hbm.at[cid, pl.ds(row0, nr), :])


def _final_body(p_ref, d_ref, bc_ref, wl_ref, bl_ref, o_ref, acc_ref):
    i = pl.program_id(0)
    nb = pl.num_programs(0)

    @pl.when(i == 0)
    def _():
        acc_ref[...] = jnp.zeros_like(acc_ref)

    blk = p_ref.shape[1]
    p = p_ref[0] + p_ref[1]
    d = d_ref[...]
    s = lax.rsqrt(jnp.maximum(d[:, 1:2] + d[:, 3:4], 1.0))
    h = jnp.maximum(p * s + bc_ref[...], 0.0)
    rows = i * blk + lax.broadcasted_iota(jnp.int32, (blk, 1), 0)
    h = jnp.where(rows < N_NODES, h, 0.0)
    acc_ref[...] += jnp.sum(h, axis=0, keepdims=True)

    @pl.when(i == nb - 1)
    def _():
        m = acc_ref[...] / float(N_NODES)
        logits = lax.dot_general(m, wl_ref[...], (((1,), (1,)), ((), ())),
                                 preferred_element_type=jnp.float32)
        logits = logits + bl_ref[...]
        e = jnp.exp(logits - jnp.max(logits))
        o_ref[...] = e / jnp.sum(e)


def kernel(in_feat, edge_index, W_conv, b_conv, W_lin, b_lin):
    ei = edge_index.astype(jnp.int32)

    deg = _deg_kernel(ei.reshape(2, NW, 1, EW))
    deg_t = deg.T

    z = pl.pallas_call(
        _scale_mm_body,
        grid=(5,),
        in_specs=[
            pl.BlockSpec((2000, F), lambda i: (i, 0)),
            pl.BlockSpec((2000, 4), lambda i: (i, 0)),
            pl.BlockSpec((F, F), lambda i: (0, 0)),
        ],
        out_specs=pl.BlockSpec((2000, F), lambda i: (i, 0)),
        out_shape=jax.ShapeDtypeStruct((N_NODES, F), jnp.float32),
    )(in_feat, deg_t[:N_NODES], W_conv)

    partials = _edge_kernel(z, ei.reshape(2, -1, 1, KE))

    BLK = 1024
    p = pl.pallas_call(
        _final_body,
        grid=(NP // BLK,),
        in_specs=[
            pl.BlockSpec((NC, BLK, F), lambda i: (0, i, 0)),
            pl.BlockSpec((BLK, 4), lambda i: (i, 0)),
            pl.BlockSpec((1, F), lambda i: (0, 0)),
            pl.BlockSpec(W_lin.shape, lambda i: (0, 0)),
            pl.BlockSpec((1, W_lin.shape[0]), lambda i: (0, 0)),
        ],
        out_specs=pl.BlockSpec((1, W_lin.shape[0]), lambda i: (0, 0)),
        out_shape=jax.ShapeDtypeStruct((1, W_lin.shape[0]), jnp.float32),
        scratch_shapes=[pltpu.VMEM((1, F), jnp.float32)],
    )(partials, deg_t, b_conv.reshape(1, F), W_lin,
      b_lin.reshape(1, W_lin.shape[0]))

    return p.reshape(W_lin.shape[0])


# speedup vs baseline: 11.5497x; 1.0909x over previous
import functools

import jax
import jax.numpy as jnp
from jax import lax
from jax.experimental import pallas as pl
from jax.experimental.pallas import tpu as pltpu
from jax.experimental.pallas import tpu_sc as plsc

NC = 2
NS = 16
L = 16
NW = NC * NS

N_NODES = 10000
N_EDGES = 320000
NP = 10240
F = 128
K = 80
EW = N_EDGES // NW
KE = 80
CH = 125
CH0 = 126
CH1 = 2 * CH - CH0
IR = 6
RB = 4

_MESH = plsc.VectorSubcoreMesh(
    core_axis_name="c", subcore_axis_name="s", num_cores=NC, num_subcores=NS)


@functools.partial(
    pl.kernel,
    out_type=jax.ShapeDtypeStruct((4, NP), jnp.float32),
    mesh=_MESH,
    scratch_types=[
        pltpu.VMEM((EW // KE, 1, KE), jnp.int32),
        pltpu.VMEM((NP,), jnp.float32),
        pltpu.VMEM((NP,), jnp.float32),
        pltpu.VMEM((NS, 1, NP // NS), jnp.float32),
        pltpu.VMEM((NP // NS,), jnp.float32),
        pltpu.VMEM_SHARED((2, NS, 1, NP), jnp.float32),
    ],
    compiler_params=pltpu.CompilerParams(needs_layout_passes=False),
)
def _deg_kernel(ei_hbm, out_hbm, idx_v, h_out, h_in, rbuf, rres,
                stage):
    cid = lax.axis_index("c")
    sid = lax.axis_index("s")
    wid = sid * NC + cid
    base = wid * EW
    ones = jnp.ones((L,), jnp.float32)
    zeros = jnp.zeros((L,), jnp.float32)

    def _zero(i, _):
        h_out[pl.ds(i * L, L)] = zeros
        h_in[pl.ds(i * L, L)] = zeros
        return _
    lax.fori_loop(0, NP // L, _zero, None)

    cw = EW // KE
    pltpu.sync_copy(ei_hbm.at[0, pl.ds(wid * cw, cw), :, :], idx_v)
    def _hist_out(i, _):
        for k in range(KE // L):
            plsc.addupdate_scatter(
                h_out, [idx_v[i, 0, pl.ds(k * L, L)]], ones)
        return _
    lax.fori_loop(0, cw, _hist_out, None)

    pltpu.sync_copy(ei_hbm.at[1, pl.ds(wid * cw, cw), :, :], idx_v)
    def _hist_in(i, _):
        for k in range(KE // L):
            plsc.addupdate_scatter(
                h_in, [idx_v[i, 0, pl.ds(k * L, L)]], ones)
        return _
    lax.fori_loop(0, cw, _hist_in, None)

    pltpu.sync_copy(h_out, stage.at[0, sid, 0, :])
    pltpu.sync_copy(h_in, stage.at[1, sid, 0, :])
    plsc.subcore_barrier()

    nc_ = NP // NS
    col0 = sid * nc_
    for r in range(2):
        pltpu.sync_copy(stage.at[r, :, pl.ds(0, 1), pl.ds(col0, nc_)], rbuf)
        def _red(i, _):
            acc = rbuf[0, 0, pl.ds(i * L, L)]
            for t in range(1, NS):
                acc = acc + rbuf[t, 0, pl.ds(i * L, L)]
            rres[pl.ds(i * L, L)] = acc
            return _
        lax.fori_loop(0, nc_ // L, _red, None)
        pltpu.sync_copy(rres, out_hbm.at[2 * cid + r, pl.ds(col0, nc_)])


def _scale_mm_body(x_ref, d_ref, w_ref, o_ref):
    d = d_ref[...]
    s = lax.rsqrt(jnp.maximum(d[:, 0:1] + d[:, 2:3], 1.0))
    o_ref[...] = jnp.dot(x_ref[...] * s, w_ref[...],
                         preferred_element_type=jnp.float32)


@functools.partial(
    pl.kernel,
    out_type=jax.ShapeDtypeStruct((NC, NP, F), jnp.float32),
    mesh=_MESH,
    scratch_types=[
        pltpu.VMEM((IR, 2, KE), jnp.int32),
        pltpu.VMEM((RB, KE, F), jnp.float32),
        pltpu.VMEM_SHARED((NP, F), jnp.float32),
        pltpu.SemaphoreType.DMA((IR,)),
        pltpu.SemaphoreType.DMA((RB,)),
        pltpu.SemaphoreType.DMA((RB,)),
    ],
)
def _edge_kernel(z_hbm, ei_hbm, out_hbm, idx_r, rows_v, acc_sh,
                 isem, gsem, ssem):
    cid = lax.axis_index("c")
    sid = lax.axis_index("s")
    base_c = jnp.where(cid == 0, sid * CH0, NS * CH0 + sid * CH1)
    nch = jnp.where(cid == 0, CH0, CH1)

    def _fill(i, _):
        r = i // (F // L)
        c = lax.rem(i, F // L)
        rows_v[0, r, pl.ds(c * L, L)] = jnp.zeros((L,), jnp.float32)
        return _
    lax.fori_loop(0, KE * (F // L), _fill, None)
    row0 = sid * (NP // NS)
    def _zero(k, _):
        pltpu.sync_copy(rows_v.at[0], acc_sh.at[pl.ds(row0 + k * KE, KE), :])
        return _
    lax.fori_loop(0, (NP // NS) // KE, _zero, None)
    plsc.subcore_barrier()

    def _idx_load(c):
        pltpu.async_copy(ei_hbm.at[0, base_c + c, 0], idx_r.at[lax.rem(c, IR), 0],
                         isem.at[lax.rem(c, IR)])
        pltpu.async_copy(ei_hbm.at[1, base_c + c, 0], idx_r.at[lax.rem(c, IR), 1],
                         isem.at[lax.rem(c, IR)])

    def _idx_wait(c):
        pltpu.make_async_copy(ei_hbm.at[0, base_c + c, 0],
                              idx_r.at[lax.rem(c, IR), 0],
                              isem.at[lax.rem(c, IR)]).wait()
        pltpu.make_async_copy(ei_hbm.at[1, base_c + c, 0],
                              idx_r.at[lax.rem(c, IR), 1],
                              isem.at[lax.rem(c, IR)]).wait()

    def _gather(c, p):
        pltpu.async_copy(z_hbm.at[idx_r.at[lax.rem(c, IR), 0]], rows_v.at[p],
                         gsem.at[p])

    def _gather_wait(c, p):
        pltpu.make_async_copy(z_hbm.at[idx_r.at[lax.rem(c, IR), 0]],
                              rows_v.at[p], gsem.at[p]).wait()

    def _scatter(c, p):
        pltpu.async_copy(rows_v.at[p], acc_sh.at[idx_r.at[lax.rem(c, IR), 1]],
                         ssem.at[p], add=True)

    def _scatter_wait(c, p):
        pltpu.make_async_copy(rows_v.at[p],
                              acc_sh.at[idx_r.at[lax.rem(c, IR), 1]],
                              ssem.at[p]).wait()

    for c in range(IR - 1):
        _idx_load(jnp.int32(c))
    for c in range(RB - 1):
        _idx_wait(jnp.int32(c))
        _gather(jnp.int32(c), c)

    def _body(j, _):
        p = lax.rem(j, RB)

        @pl.when(j > 0)
        def _():
            _scatter_wait(j - 1, lax.rem(j - 1, RB))

        @pl.when(j + IR - 1 < nch)
        def _():
            _idx_load(j + IR - 1)

        @pl.when(j + RB - 1 < nch)
        def _():
            _idx_wait(j + RB - 1)
            _gather(j + RB - 1, lax.rem(j + RB - 1, RB))

        _gather_wait(j, p)
        _scatter(j, p)
        return _
    lax.fori_loop(0, nch, _body, None)
    _scatter_wait(nch - 1, lax.rem(nch - 1, RB))
    plsc.subcore_barrier()

    nr = NP // NS
    pltpu.sync_copy(acc_sh.at[pl.ds(row0, nr), :],
                    out_hbm.at[cid, pl.ds(row0, nr), :])


def _final_body(p_ref, d_ref, bc_ref, wl_ref, bl_ref, o_ref, acc_ref):
    i = pl.program_id(0)
    nb = pl.num_programs(0)

    @pl.when(i == 0)
    def _():
        acc_ref[...] = jnp.zeros_like(acc_ref)

    blk = p_ref.shape[1]
    p = p_ref[0] + p_ref[1]
    d = d_ref[...]
    s = lax.rsqrt(jnp.maximum(d[:, 1:2] + d[:, 3:4], 1.0))
    h = jnp.maximum(p * s + bc_ref[...], 0.0)
    rows = i * blk + lax.broadcasted_iota(jnp.int32, (blk, 1), 0)
    h = jnp.where(rows < N_NODES, h, 0.0)
    acc_ref[...] += jnp.sum(h, axis=0, keepdims=True)

    @pl.when(i == nb - 1)
    def _():
        m = acc_ref[...] / float(N_NODES)
        logits = lax.dot_general(m, wl_ref[...], (((1,), (1,)), ((), ())),
                                 preferred_element_type=jnp.float32)
        logits = logits + bl_ref[...]
        e = jnp.exp(logits - jnp.max(logits))
        o_ref[...] = e / jnp.sum(e)


def kernel(in_feat, edge_index, W_conv, b_conv, W_lin, b_lin):
    ei = edge_index.astype(jnp.int32)

    ei4 = ei.reshape(2, -1, 1, KE)
    deg = _deg_kernel(ei4)
    deg_t = deg.T

    z = pl.pallas_call(
        _scale_mm_body,
        grid=(5,),
        in_specs=[
            pl.BlockSpec((2000, F), lambda i: (i, 0)),
            pl.BlockSpec((2000, 4), lambda i: (i, 0)),
            pl.BlockSpec((F, F), lambda i: (0, 0)),
        ],
        out_specs=pl.BlockSpec((2000, F), lambda i: (i, 0)),
        out_shape=jax.ShapeDtypeStruct((N_NODES, F), jnp.float32),
    )(in_feat, deg_t, W_conv)

    partials = _edge_kernel(z, ei4)

    BLK = 2048
    p = pl.pallas_call(
        _final_body,
        grid=(NP // BLK,),
        in_specs=[
            pl.BlockSpec((NC, BLK, F), lambda i: (0, i, 0)),
            pl.BlockSpec((BLK, 4), lambda i: (i, 0)),
            pl.BlockSpec((1, F), lambda i: (0, 0)),
            pl.BlockSpec(W_lin.shape, lambda i: (0, 0)),
            pl.BlockSpec((1, W_lin.shape[0]), lambda i: (0, 0)),
        ],
        out_specs=pl.BlockSpec((1, W_lin.shape[0]), lambda i: (0, 0)),
        out_shape=jax.ShapeDtypeStruct((1, W_lin.shape[0]), jnp.float32),
        scratch_shapes=[pltpu.VMEM((1, F), jnp.float32)],
    )(partials, deg_t, b_conv.reshape(1, F), W_lin,
      b_lin.reshape(1, W_lin.shape[0]))

    return p.reshape(W_lin.shape[0])


# speedup vs baseline: 12.1949x; 1.0559x over previous
import functools

import jax
import jax.numpy as jnp
from jax import lax
from jax.experimental import pallas as pl
from jax.experimental.pallas import tpu as pltpu
from jax.experimental.pallas import tpu_sc as plsc

NC = 2
NS = 16
L = 16
NW = NC * NS

N_NODES = 10000
N_EDGES = 320000
NP = 10240
F = 128
K = 80
EW = N_EDGES // NW
KE = 80
CH = 125
CH0 = 126
CH1 = 2 * CH - CH0
IR = 6
RB = 4

_MESH = plsc.VectorSubcoreMesh(
    core_axis_name="c", subcore_axis_name="s", num_cores=NC, num_subcores=NS)


@functools.partial(
    pl.kernel,
    out_type=jax.ShapeDtypeStruct((4, NP), jnp.float32),
    mesh=_MESH,
    scratch_types=[
        pltpu.VMEM((EW // KE, 1, KE), jnp.int32),
        pltpu.VMEM((EW // KE, 1, KE), jnp.int32),
        pltpu.SemaphoreType.DMA,
        pltpu.VMEM((NP,), jnp.float32),
        pltpu.VMEM((NP,), jnp.float32),
        pltpu.VMEM((NS, 1, NP // NS), jnp.float32),
        pltpu.VMEM((NP // NS,), jnp.float32),
        pltpu.VMEM_SHARED((2, NS, 1, NP), jnp.float32),
    ],
    compiler_params=pltpu.CompilerParams(needs_layout_passes=False),
)
def _deg_kernel(ei_hbm, out_hbm, idx_v, idx_w, dsem, h_out, h_in, rbuf,
                rres, stage):
    cid = lax.axis_index("c")
    sid = lax.axis_index("s")
    wid = sid * NC + cid
    base = wid * EW
    ones = jnp.ones((L,), jnp.float32)
    zeros = jnp.zeros((L,), jnp.float32)

    def _zero(i, _):
        for k in range(8):
            h_out[pl.ds((i * 8 + k) * L, L)] = zeros
            h_in[pl.ds((i * 8 + k) * L, L)] = zeros
        return _
    lax.fori_loop(0, NP // (8 * L), _zero, None)

    cw = EW // KE
    dcp = pltpu.async_copy(ei_hbm.at[1, pl.ds(wid * cw, cw), :, :], idx_w,
                           dsem)
    pltpu.sync_copy(ei_hbm.at[0, pl.ds(wid * cw, cw), :, :], idx_v)
    def _hist_out(i, _):
        for k in range(KE // L):
            plsc.addupdate_scatter(
                h_out, [idx_v[i, 0, pl.ds(k * L, L)]], ones)
        return _
    lax.fori_loop(0, cw, _hist_out, None)

    dcp.wait()
    def _hist_in(i, _):
        for k in range(KE // L):
            plsc.addupdate_scatter(
                h_in, [idx_w[i, 0, pl.ds(k * L, L)]], ones)
        return _
    lax.fori_loop(0, cw, _hist_in, None)

    pltpu.sync_copy(h_out, stage.at[0, sid, 0, :])
    pltpu.sync_copy(h_in, stage.at[1, sid, 0, :])
    plsc.subcore_barrier()

    nc_ = NP // NS
    col0 = sid * nc_
    for r in range(2):
        pltpu.sync_copy(stage.at[r, :, pl.ds(0, 1), pl.ds(col0, nc_)], rbuf)
        def _red(i, _):
            acc = rbuf[0, 0, pl.ds(i * L, L)]
            for t in range(1, NS):
                acc = acc + rbuf[t, 0, pl.ds(i * L, L)]
            rres[pl.ds(i * L, L)] = acc
            return _
        lax.fori_loop(0, nc_ // L, _red, None)
        pltpu.sync_copy(rres, out_hbm.at[2 * cid + r, pl.ds(col0, nc_)])


def _scale_mm_body(x_ref, d_ref, w_ref, o_ref):
    d = jnp.transpose(d_ref[...])
    s = lax.rsqrt(jnp.maximum(d[:, 0:1] + d[:, 2:3], 1.0))
    o_ref[...] = jnp.dot(x_ref[...].astype(jnp.bfloat16) * s.astype(jnp.bfloat16),
                         w_ref[...], preferred_element_type=jnp.float32)


@functools.partial(
    pl.kernel,
    out_type=jax.ShapeDtypeStruct((NC, NP, F), jnp.float32),
    mesh=_MESH,
    scratch_types=[
        pltpu.VMEM((IR, 2, KE), jnp.int32),
        pltpu.VMEM((RB, KE, F), jnp.float32),
        pltpu.VMEM_SHARED((NP, F), jnp.float32),
        pltpu.SemaphoreType.DMA((IR,)),
        pltpu.SemaphoreType.DMA((RB,)),
        pltpu.SemaphoreType.DMA((RB,)),
    ],
)
def _edge_kernel(z_hbm, ei_hbm, out_hbm, idx_r, rows_v, acc_sh,
                 isem, gsem, ssem):
    cid = lax.axis_index("c")
    sid = lax.axis_index("s")
    base_c = jnp.where(cid == 0, sid * CH0, NS * CH0 + sid * CH1)
    nch = jnp.where(cid == 0, CH0, CH1)

    def _fill(i, _):
        r = i // (F // L)
        c = lax.rem(i, F // L)
        rows_v[0, r, pl.ds(c * L, L)] = jnp.zeros((L,), jnp.float32)
        return _
    lax.fori_loop(0, KE * (F // L), _fill, None)
    row0 = sid * (NP // NS)
    def _zero(k, _):
        pltpu.async_copy(rows_v.at[0], acc_sh.at[pl.ds(row0 + k * KE, KE), :],
                         gsem.at[0])
        return _
    lax.fori_loop(0, (NP // NS) // KE, _zero, None)
    def _zwait(k, _):
        pltpu.make_async_copy(rows_v.at[0],
                              acc_sh.at[pl.ds(row0 + k * KE, KE), :],
                              gsem.at[0]).wait()
        return _
    lax.fori_loop(0, (NP // NS) // KE, _zwait, None)
    plsc.subcore_barrier()

    def _idx_load(c):
        pltpu.async_copy(ei_hbm.at[0, base_c + c, 0], idx_r.at[lax.rem(c, IR), 0],
                         isem.at[lax.rem(c, IR)])
        pltpu.async_copy(ei_hbm.at[1, base_c + c, 0], idx_r.at[lax.rem(c, IR), 1],
                         isem.at[lax.rem(c, IR)])

    def _idx_wait(c):
        pltpu.make_async_copy(ei_hbm.at[0, base_c + c, 0],
                              idx_r.at[lax.rem(c, IR), 0],
                              isem.at[lax.rem(c, IR)]).wait()
        pltpu.make_async_copy(ei_hbm.at[1, base_c + c, 0],
                              idx_r.at[lax.rem(c, IR), 1],
                              isem.at[lax.rem(c, IR)]).wait()

    def _gather(c, p):
        pltpu.async_copy(z_hbm.at[idx_r.at[lax.rem(c, IR), 0]], rows_v.at[p],
                         gsem.at[p])

    def _gather_wait(c, p):
        pltpu.make_async_copy(z_hbm.at[idx_r.at[lax.rem(c, IR), 0]],
                              rows_v.at[p], gsem.at[p]).wait()

    def _scatter(c, p):
        pltpu.async_copy(rows_v.at[p], acc_sh.at[idx_r.at[lax.rem(c, IR), 1]],
                         ssem.at[p], add=True)

    def _scatter_wait(c, p):
        pltpu.make_async_copy(rows_v.at[p],
                              acc_sh.at[idx_r.at[lax.rem(c, IR), 1]],
                              ssem.at[p]).wait()

    for c in range(IR - 1):
        _idx_load(jnp.int32(c))
    for c in range(RB - 1):
        _idx_wait(jnp.int32(c))
        _gather(jnp.int32(c), c)

    def _body(j, _):
        p = lax.rem(j, RB)

        @pl.when(j > 0)
        def _():
            _scatter_wait(j - 1, lax.rem(j - 1, RB))

        @pl.when(j + IR - 1 < nch)
        def _():
            _idx_load(j + IR - 1)

        @pl.when(j + RB - 1 < nch)
        def _():
            _idx_wait(j + RB - 1)
            _gather(j + RB - 1, lax.rem(j + RB - 1, RB))

        _gather_wait(j, p)
        _scatter(j, p)
        return _
    lax.fori_loop(0, nch, _body, None)
    _scatter_wait(nch - 1, lax.rem(nch - 1, RB))
    plsc.subcore_barrier()

    nr = NP // NS
    pltpu.sync_copy(acc_sh.at[pl.ds(row0, nr), :],
                    out_hbm.at[cid, pl.ds(row0, nr), :])


def _final_body(p_ref, d_ref, bc_ref, wl_ref, bl_ref, o_ref, acc_ref):
    i = pl.program_id(0)
    nb = pl.num_programs(0)

    @pl.when(i == 0)
    def _():
        acc_ref[...] = jnp.zeros_like(acc_ref)

    blk = p_ref.shape[1]
    p = p_ref[0] + p_ref[1]
    d = jnp.transpose(d_ref[...])
    s = lax.rsqrt(jnp.maximum(d[:, 1:2] + d[:, 3:4], 1.0))
    h = jnp.maximum(p * s + bc_ref[...], 0.0)
    rows = i * blk + lax.broadcasted_iota(jnp.int32, (blk, 1), 0)
    h = jnp.where(rows < N_NODES, h, 0.0)
    acc_ref[...] += jnp.sum(h, axis=0, keepdims=True)

    @pl.when(i == nb - 1)
    def _():
        m = acc_ref[...] / float(N_NODES)
        logits = lax.dot_general(m, wl_ref[...], (((1,), (1,)), ((), ())),
                                 preferred_element_type=jnp.float32)
        logits = logits + bl_ref[...]
        e = jnp.exp(logits - jnp.max(logits))
        o_ref[...] = e / jnp.sum(e)


def kernel(in_feat, edge_index, W_conv, b_conv, W_lin, b_lin):
    ei = edge_index.astype(jnp.int32)

    ei4 = ei.reshape(2, -1, 1, KE)
    deg = _deg_kernel(ei4)

    z = pl.pallas_call(
        _scale_mm_body,
        grid=(5,),
        in_specs=[
            pl.BlockSpec((2048, F), lambda i: (i, 0)),
            pl.BlockSpec((4, 2048), lambda i: (0, i)),
            pl.BlockSpec((F, F), lambda i: (0, 0)),
        ],
        out_specs=pl.BlockSpec((2048, F), lambda i: (i, 0)),
        out_shape=jax.ShapeDtypeStruct((N_NODES, F), jnp.float32),
    )(in_feat, deg, W_conv.astype(jnp.bfloat16))

    partials = _edge_kernel(z, ei4)

    BLK = 2048
    p = pl.pallas_call(
        _final_body,
        grid=(NP // BLK,),
        in_specs=[
            pl.BlockSpec((NC, BLK, F), lambda i: (0, i, 0)),
            pl.BlockSpec((4, BLK), lambda i: (0, i)),
            pl.BlockSpec((1, F), lambda i: (0, 0)),
            pl.BlockSpec(W_lin.shape, lambda i: (0, 0)),
            pl.BlockSpec((1, W_lin.shape[0]), lambda i: (0, 0)),
        ],
        out_specs=pl.BlockSpec((1, W_lin.shape[0]), lambda i: (0, 0)),
        out_shape=jax.ShapeDtypeStruct((1, W_lin.shape[0]), jnp.float32),
        scratch_shapes=[pltpu.VMEM((1, F), jnp.float32)],
    )(partials, deg, b_conv.reshape(1, F), W_lin,
      b_lin.reshape(1, W_lin.shape[0]))

    return p.reshape(W_lin.shape[0])
